# Initial kernel scaffold; baseline (speedup 1.0000x reference)
import jax
import jax.numpy as jnp
from jax.experimental import pallas as pl


def kernel(ids, weight):
    raise NotImplementedError("write your pallas kernel here")


# speedup vs baseline: 1.4363x; 1.4363x over previous
import functools

import jax
import jax.numpy as jnp
from jax import lax
from jax.experimental import pallas as pl
from jax.experimental.pallas import tpu as pltpu
from jax.experimental.pallas import tpu_sc as plsc

_NW = 32
_CHUNK = 128


@functools.cache
def _make_gather(V, D, N):
    bpw = N // _NW
    nchunks = bpw // _CHUNK
    mesh = plsc.VectorSubcoreMesh(core_axis_name="c", subcore_axis_name="s")

    @functools.partial(
        pl.kernel,
        out_type=jax.ShapeDtypeStruct((N, D), jnp.float32),
        mesh=mesh,
        compiler_params=pltpu.CompilerParams(use_tc_tiling_on_sc=False),
        scratch_types=[
            pltpu.VMEM((nchunks, _CHUNK), jnp.int32),
            pltpu.VMEM((_CHUNK, D), jnp.float32),
            pltpu.SemaphoreType.DMA,
        ],
    )
    def gather_kernel(table, idx, out, idx_v, rows_v, sem):
        wid = lax.axis_index("s") * 2 + lax.axis_index("c")
        base = wid * bpw
        pltpu.sync_copy(idx.at[wid], idx_v)

        def body(j, carry):
            pltpu.async_copy(table.at[idx_v.at[j]], rows_v, sem).wait()
            pltpu.sync_copy(rows_v, out.at[pl.ds(base + j * _CHUNK, _CHUNK)])
            return carry

        lax.fori_loop(0, nchunks, body, 0)

    return gather_kernel


def kernel(ids, weight):
    B, F = ids.shape
    V, D = weight.shape
    N = B * F
    idx = ids.astype(jnp.int32).reshape(_NW, N // _NW // _CHUNK, _CHUNK)
    out = _make_gather(V, D, N)(weight, idx)
    return out.reshape(B, F, D)


# speedup vs baseline: 1.5395x; 1.0718x over previous
import functools

import jax
import jax.numpy as jnp
from jax import lax
from jax.experimental import pallas as pl
from jax.experimental.pallas import tpu as pltpu
from jax.experimental.pallas import tpu_sc as plsc

_NW = 32
_CHUNK = 512


@functools.cache
def _make_gather(V, D, N):
    bpw = N // _NW
    nchunks = bpw // _CHUNK
    mesh = plsc.VectorSubcoreMesh(core_axis_name="c", subcore_axis_name="s")

    @functools.partial(
        pl.kernel,
        out_type=jax.ShapeDtypeStruct((N, D), jnp.float32),
        mesh=mesh,
        compiler_params=pltpu.CompilerParams(use_tc_tiling_on_sc=False),
        scratch_types=[
            pltpu.VMEM((nchunks, _CHUNK), jnp.int32),
            pltpu.VMEM((_CHUNK, D), jnp.float32),
            pltpu.SemaphoreType.DMA,
        ],
    )
    def gather_kernel(table, idx, out, idx_v, rows_v, sem):
        wid = lax.axis_index("s") * 2 + lax.axis_index("c")
        base = wid * bpw
        pltpu.sync_copy(idx.at[wid], idx_v)

        def body(j, carry):
            pltpu.async_copy(table.at[idx_v.at[j]], rows_v, sem).wait()
            pltpu.sync_copy(rows_v, out.at[pl.ds(base + j * _CHUNK, _CHUNK)])
            return carry

        lax.fori_loop(0, nchunks, body, 0)

    return gather_kernel


def kernel(ids, weight):
    B, F = ids.shape
    V, D = weight.shape
    N = B * F
    idx = ids.astype(jnp.int32).reshape(_NW, N // _NW // _CHUNK, _CHUNK)
    out = _make_gather(V, D, N)(weight, idx)
    return out.reshape(B, F, D)


# speedup vs baseline: 1.5763x; 1.0239x over previous
import functools

import jax
import jax.numpy as jnp
from jax import lax
from jax.experimental import pallas as pl
from jax.experimental.pallas import tpu as pltpu
from jax.experimental.pallas import tpu_sc as plsc

_NW = 32
_CHUNK = 256
_NBUF = 4
_H = 2


@functools.cache
def _make_gather(V, D, N):
    bpw = N // _NW
    nchunks = bpw // _CHUNK
    nsuper = nchunks // _NBUF
    mesh = plsc.VectorSubcoreMesh(core_axis_name="c", subcore_axis_name="s")

    @functools.partial(
        pl.kernel,
        out_type=jax.ShapeDtypeStruct((N, D), jnp.float32),
        mesh=mesh,
        compiler_params=pltpu.CompilerParams(use_tc_tiling_on_sc=False),
        scratch_types=[
            pltpu.VMEM((nchunks, _CHUNK), jnp.int32),
            pltpu.VMEM((_NBUF, _CHUNK, D), jnp.float32),
            [pltpu.SemaphoreType.DMA] * _NBUF,
            [pltpu.SemaphoreType.DMA] * _NBUF,
        ],
    )
    def gather_kernel(table, idx, out, idx_v, rows_v, gs, os):
        wid = lax.axis_index("s") * 2 + lax.axis_index("c")
        base = wid * bpw
        pltpu.sync_copy(idx.at[wid], idx_v)

        def gather_desc(j, b):
            return pltpu.make_async_copy(table.at[idx_v.at[j]], rows_v.at[b], gs[b])

        def write_desc(j, b):
            return pltpu.make_async_copy(
                rows_v.at[b], out.at[pl.ds(base + j * _CHUNK, _CHUNK)], os[b]
            )

        for j in range(_NBUF - _H):
            gather_desc(j, j % _NBUF).start()

        def super_body(sg, carry):
            for b in range(_NBUF):
                j = sg * _NBUF + b
                bw = (b - _H) % _NBUF
                jn = j - _H + _NBUF

                @pl.when(j >= _H)
                def _():
                    write_desc(j - _H, bw).wait()

                @pl.when(jn < nchunks)
                def _():
                    gather_desc(jn, bw).start()

                gather_desc(j, b).wait()
                write_desc(j, b).start()
            return carry

        lax.fori_loop(0, nsuper, super_body, 0)
        for j in range(nchunks - _H, nchunks):
            write_desc(j, j % _NBUF).wait()

    return gather_kernel


def kernel(ids, weight):
    B, F = ids.shape
    V, D = weight.shape
    N = B * F
    idx = ids.astype(jnp.int32).reshape(_NW, N // _NW // _CHUNK, _CHUNK)
    out = _make_gather(V, D, N)(weight, idx)
    return out.reshape(B, F, D)
